# Initial kernel scaffold; baseline (speedup 1.0000x reference)
import jax
import jax.numpy as jnp
from jax.experimental import pallas as pl


def kernel(rel_bias, L):
    raise NotImplementedError("write your pallas kernel here")


# speedup vs baseline: 40.1202x; 40.1202x over previous
import functools

import jax
import jax.numpy as jnp
from jax import lax
from jax.experimental import pallas as pl
from jax.experimental.pallas import tpu as pltpu
from jax.experimental.pallas import tpu_sc as plsc

_H = 16
_L = 2048
_ROWS = 16
_BLOCKS = _L // _ROWS
_HALF_BLOCKS = _BLOCKS // 2


def _sc_fill_body(slabs_hbm, out_hbm, dslab_v):
    c = lax.axis_index("c")
    s = lax.axis_index("s")
    h = s
    pltpu.sync_copy(slabs_hbm.at[h], dslab_v)

    def blk(t, carry):
        r0 = (c * _HALF_BLOCKS + t) * _ROWS
        start = _L - r0
        pltpu.sync_copy(dslab_v.at[:, pl.ds(start, _L)],
                        out_hbm.at[h, pl.ds(r0, _ROWS), :])
        return carry

    lax.fori_loop(0, _HALF_BLOCKS, blk, 0)


_sc_fill = functools.partial(
    pl.kernel,
    out_type=jax.ShapeDtypeStruct((_H, _L, _L), jnp.float32),
    scratch_types=[pltpu.VMEM((_ROWS, 2 * _L), jnp.float32)],
    mesh=plsc.VectorSubcoreMesh(core_axis_name="c", subcore_axis_name="s"),
    compiler_params=pltpu.CompilerParams(use_tc_tiling_on_sc=False),
)(_sc_fill_body)


def kernel(rel_bias, L):
    del L
    dext = jnp.tile(rel_bias[:, ::-1], (1, 3))
    slabs = jnp.stack(
        [lax.slice_in_dim(dext, _L - 1 - a, _L - 1 - a + 2 * _L, axis=1)
         for a in range(_ROWS)], axis=1)
    return _sc_fill(slabs)
